# baseline (speedup 1.0000x reference)
import functools

import jax
import jax.numpy as jnp
from jax import lax
from jax.experimental import pallas as pl
from jax.experimental.pallas import tpu as pltpu
from jax.experimental.pallas import tpu_sc as plsc

N_H = 64
NC, NS = 2, 16
NW = NC * NS
CHUNK = 128


@functools.lru_cache(maxsize=None)
def _gather_kernel(n_rows):
    n_per_w = n_rows // NW
    n_chunks = n_per_w // CHUNK
    mesh = plsc.VectorSubcoreMesh(core_axis_name="c", subcore_axis_name="s")

    @functools.partial(
        pl.kernel,
        out_type=jax.ShapeDtypeStruct((NW, n_chunks, CHUNK, N_H), jnp.float32),
        mesh=mesh,
        scratch_types=[
            pltpu.VMEM((n_chunks, CHUNK), jnp.int32),
            pltpu.VMEM((CHUNK, N_H), jnp.float32),
            pltpu.SemaphoreType.DMA,
        ],
        compiler_params=pltpu.CompilerParams(use_tc_tiling_on_sc=False),
    )
    def k(idx_hbm, table_hbm, out_hbm, idx_v, rows_v, sem):
        wid = lax.axis_index("s") * NC + lax.axis_index("c")
        pltpu.sync_copy(idx_hbm.at[wid], idx_v)

        def body(j, carry):
            pltpu.async_copy(table_hbm.at[idx_v.at[j]], rows_v, sem).wait()
            pltpu.sync_copy(rows_v, out_hbm.at[wid, j])
            return carry

        lax.fori_loop(0, n_chunks, body, 0)

    return k


def kernel(inputs, V, b):
    B, S = inputs.shape
    n_rows = B * S
    idx = inputs.reshape(NW, n_rows // NW // CHUNK, CHUNK).astype(jnp.int32)
    out = _gather_kernel(n_rows)(idx, V)
    return out.reshape(B, S, N_H)


# speedup vs baseline: 1.0052x; 1.0052x over previous
import functools

import jax
import jax.numpy as jnp
from jax import lax
from jax.experimental import pallas as pl
from jax.experimental.pallas import tpu as pltpu
from jax.experimental.pallas import tpu_sc as plsc

N_H = 64
PADW = 128
NC, NS = 2, 16
NW = NC * NS
CHUNK = 128


@functools.lru_cache(maxsize=None)
def _gather_kernel(n_rows):
    n_per_w = n_rows // NW
    n_chunks = n_per_w // CHUNK
    mesh = plsc.VectorSubcoreMesh(core_axis_name="c", subcore_axis_name="s")

    @functools.partial(
        pl.kernel,
        out_type=jax.ShapeDtypeStruct((NW, n_chunks, CHUNK, PADW), jnp.float32),
        mesh=mesh,
        scratch_types=[
            pltpu.VMEM((n_chunks, CHUNK), jnp.int32),
            pltpu.VMEM((CHUNK, PADW), jnp.float32),
            pltpu.SemaphoreType.DMA,
        ],
        compiler_params=pltpu.CompilerParams(use_tc_tiling_on_sc=True),
    )
    def k(idx_hbm, table_hbm, out_hbm, idx_v, rows_v, sem):
        wid = lax.axis_index("s") * NC + lax.axis_index("c")
        pltpu.sync_copy(idx_hbm.at[wid], idx_v)

        def body(j, carry):
            pltpu.async_copy(table_hbm.at[idx_v.at[j]], rows_v, sem).wait()
            pltpu.sync_copy(rows_v, out_hbm.at[wid, j])
            return carry

        lax.fori_loop(0, n_chunks, body, 0)

    return k


def kernel(inputs, V, b):
    B, S = inputs.shape
    n_rows = B * S
    Vp = jnp.pad(V, ((0, 0), (0, PADW - N_H)))
    idx = inputs.reshape(NW, n_rows // NW // CHUNK, CHUNK).astype(jnp.int32)
    out = _gather_kernel(n_rows)(idx, Vp)
    return out[..., :N_H].reshape(B, S, N_H)
